# baseline (speedup 1.0000x reference)
import functools
import jax
import jax.numpy as jnp
from jax import lax
from jax.experimental import pallas as pl
from jax.experimental.pallas import tpu as pltpu
from jax.experimental.pallas import tpu_sc as plsc

NC, NS, L = 2, 16, 16
NW = NC * NS
B = 16384
D = 64
BPW = B // NW
CH = 128
NCHUNK = BPW // CH

_mesh = plsc.VectorSubcoreMesh(core_axis_name="c", subcore_axis_name="s")


@functools.partial(
    pl.kernel,
    out_type=jax.ShapeDtypeStruct((B,), jnp.float32),
    mesh=_mesh,
    compiler_params=pltpu.CompilerParams(
        needs_layout_passes=False, use_tc_tiling_on_sc=True),
    scratch_types=[
        pltpu.VMEM((NCHUNK, CH), jnp.int32),
        pltpu.VMEM((NCHUNK, CH), jnp.int32),
        pltpu.VMEM((NCHUNK, CH), jnp.int32),
        pltpu.VMEM((NCHUNK, CH), jnp.int32),
        pltpu.VMEM((CH, 2 * D), jnp.float32),
        pltpu.VMEM((CH, 2 * D), jnp.float32),
        pltpu.VMEM((CH, 2 * D), jnp.float32),
        pltpu.VMEM((CH, 2 * D), jnp.float32),
        pltpu.VMEM((BPW,), jnp.float32),
        pltpu.SemaphoreType.DMA,
        pltpu.SemaphoreType.DMA,
    ],
)
def _two_tower(u_hbm, p_hbm, ut_hbm, pt_hbm, out_hbm,
               u_idx, p_idx, u_sh, p_sh,
               u_buf0, u_buf1, p_buf0, p_buf1,
               out_v, sem0, sem1):
    wid = lax.axis_index("s") * NC + lax.axis_index("c")
    base = wid * BPW
    ubufs = (u_buf0, u_buf1)
    pbufs = (p_buf0, p_buf1)
    sems = (sem0, sem1)

    for c in range(NCHUNK):
        pltpu.sync_copy(u_hbm.at[pl.ds(base + c * CH, CH)], u_idx.at[c])
        pltpu.sync_copy(p_hbm.at[pl.ds(base + c * CH, CH)], p_idx.at[c])

    for c in range(NCHUNK):
        def shift_c(j, carry, c=c):
            uv = u_idx[c, pl.ds(j * L, L)]
            pv = p_idx[c, pl.ds(j * L, L)]
            u_sh[c, pl.ds(j * L, L)] = uv >> 1
            p_sh[c, pl.ds(j * L, L)] = pv >> 1
            return carry
        lax.fori_loop(0, CH // L, shift_c, 0)

    def fire(c):
        bb = c % 2
        du = pltpu.async_copy(ut_hbm.at[u_sh.at[c]], ubufs[bb], sems[bb])
        dp = pltpu.async_copy(pt_hbm.at[p_sh.at[c]], pbufs[bb], sems[bb])
        return du, dp

    lanes = lax.iota(jnp.int32, L)

    def compute(c):
        bb = c % 2
        ub, pb = ubufs[bb], pbufs[bb]

        def group(g, carry):
            rows = lanes + g * L
            off_u = (u_idx[c, pl.ds(g * L, L)] & 1) * D
            off_p = (p_idx[c, pl.ds(g * L, L)] & 1) * D
            acc = jnp.zeros((L,), jnp.float32)
            for d in range(D):
                ug = plsc.load_gather(ub, [rows, off_u + d])
                pg = plsc.load_gather(pb, [rows, off_p + d])
                acc = acc + ug * pg
            res = 1.0 / (1.0 + jnp.exp(-acc))
            out_v[pl.ds(c * CH + g * L, L)] = res
            return carry

        lax.fori_loop(0, CH // L, group, 0)

    pending = [fire(0), fire(1)]
    for c in range(NCHUNK):
        du, dp = pending[c]
        du.wait()
        dp.wait()
        compute(c)
        if c + 2 < NCHUNK:
            pending.append(fire(c + 2))

    pltpu.sync_copy(out_v, out_hbm.at[pl.ds(base, BPW)])


def _rowmajor_view(table):
    n = table.shape[0]
    t3 = table.T.reshape(D, n // 2, 2)
    return jnp.transpose(t3, (1, 2, 0)).reshape(n // 2, 2 * D)


def kernel(u, p, user_table, prod_table):
    return _two_tower(u, p, _rowmajor_view(user_table),
                      _rowmajor_view(prod_table))


# speedup vs baseline: 1.2642x; 1.2642x over previous
import functools
import jax
import jax.numpy as jnp
from jax import lax
from jax.experimental import pallas as pl
from jax.experimental.pallas import tpu as pltpu
from jax.experimental.pallas import tpu_sc as plsc

NC, NS, L = 2, 16, 16
NW = NC * NS
B = 16384
D = 64
DP = 128
BPW = B // NW
CH = 128
NCHUNK = BPW // CH

_mesh = plsc.VectorSubcoreMesh(core_axis_name="c", subcore_axis_name="s")


@functools.partial(
    pl.kernel,
    out_type=jax.ShapeDtypeStruct((B,), jnp.float32),
    mesh=_mesh,
    compiler_params=pltpu.CompilerParams(
        needs_layout_passes=False, use_tc_tiling_on_sc=True),
    scratch_types=[
        pltpu.VMEM((NCHUNK, CH), jnp.int32),
        pltpu.VMEM((NCHUNK, CH), jnp.int32),
        pltpu.VMEM((CH, DP), jnp.float32),
        pltpu.VMEM((CH, DP), jnp.float32),
        pltpu.VMEM((CH, DP), jnp.float32),
        pltpu.VMEM((CH, DP), jnp.float32),
        pltpu.VMEM((BPW,), jnp.float32),
        pltpu.SemaphoreType.DMA,
        pltpu.SemaphoreType.DMA,
    ],
)
def _two_tower(u_hbm, p_hbm, ut_hbm, pt_hbm, out_hbm,
               u_idx, p_idx,
               u_buf0, u_buf1, p_buf0, p_buf1,
               out_v, sem0, sem1):
    wid = lax.axis_index("s") * NC + lax.axis_index("c")
    base = wid * BPW
    ubufs = (u_buf0, u_buf1)
    pbufs = (p_buf0, p_buf1)
    sems = (sem0, sem1)

    for c in range(NCHUNK):
        pltpu.sync_copy(u_hbm.at[pl.ds(base + c * CH, CH)], u_idx.at[c])
        pltpu.sync_copy(p_hbm.at[pl.ds(base + c * CH, CH)], p_idx.at[c])

    def fire(c):
        bb = c % 2
        du = pltpu.async_copy(ut_hbm.at[u_idx.at[c]], ubufs[bb], sems[bb])
        dp = pltpu.async_copy(pt_hbm.at[p_idx.at[c]], pbufs[bb], sems[bb])
        return du, dp

    lanes = lax.iota(jnp.int32, L)

    def compute(c):
        bb = c % 2
        ub, pb = ubufs[bb], pbufs[bb]

        def group(g, carry):
            rows = lanes + g * L
            acc = jnp.zeros((L,), jnp.float32)
            for d in range(D):
                col = jnp.full((L,), d, jnp.int32)
                ug = plsc.load_gather(ub, [rows, col])
                pg = plsc.load_gather(pb, [rows, col])
                acc = acc + ug * pg
            res = 1.0 / (1.0 + jnp.exp(-acc))
            out_v[pl.ds(c * CH + g * L, L)] = res
            return carry

        lax.fori_loop(0, CH // L, group, 0)

    pending = [fire(0), fire(1)]
    for c in range(NCHUNK):
        du, dp = pending[c]
        du.wait()
        dp.wait()
        compute(c)
        if c + 2 < NCHUNK:
            pending.append(fire(c + 2))

    pltpu.sync_copy(out_v, out_hbm.at[pl.ds(base, BPW)])


def kernel(u, p, user_table, prod_table):
    utp = jnp.pad(user_table, ((0, 0), (0, DP - D)))
    ptp = jnp.pad(prod_table, ((0, 0), (0, DP - D)))
    return _two_tower(u, p, utp, ptp)


# speedup vs baseline: 1.4323x; 1.1329x over previous
import functools
import jax
import jax.numpy as jnp
from jax import lax
from jax.experimental import pallas as pl
from jax.experimental.pallas import tpu as pltpu
from jax.experimental.pallas import tpu_sc as plsc

NC, NS, L = 2, 16, 16
NW = NC * NS
B = 16384
D = 64
DP = 128
BPW = B // NW
CH = 128
NCHUNK = BPW // CH

_mesh = plsc.VectorSubcoreMesh(core_axis_name="c", subcore_axis_name="s")


@functools.partial(
    pl.kernel,
    out_type=jax.ShapeDtypeStruct((B,), jnp.float32),
    mesh=_mesh,
    compiler_params=pltpu.CompilerParams(
        needs_layout_passes=False, use_tc_tiling_on_sc=True),
    scratch_types=[
        pltpu.VMEM((NCHUNK, CH), jnp.int32),
        pltpu.VMEM((NCHUNK, CH), jnp.int32),
        pltpu.VMEM((CH, DP), jnp.float32),
        pltpu.VMEM((CH, DP), jnp.float32),
        pltpu.VMEM((CH, DP), jnp.float32),
        pltpu.VMEM((CH, DP), jnp.float32),
        pltpu.VMEM((BPW,), jnp.float32),
        pltpu.SemaphoreType.DMA,
        pltpu.SemaphoreType.DMA,
    ],
)
def _two_tower(u_hbm, p_hbm, c_hbm, out_hbm,
               u_idx, p_idx,
               u_buf0, u_buf1, p_buf0, p_buf1,
               out_v, sem0, sem1):
    wid = lax.axis_index("s") * NC + lax.axis_index("c")
    base = wid * BPW
    ubufs = (u_buf0, u_buf1)
    pbufs = (p_buf0, p_buf1)
    sems = (sem0, sem1)

    for c in range(NCHUNK):
        pltpu.sync_copy(u_hbm.at[pl.ds(base + c * CH, CH)], u_idx.at[c])
        pltpu.sync_copy(p_hbm.at[pl.ds(base + c * CH, CH)], p_idx.at[c])

    def fire(c):
        bb = c % 2
        du = pltpu.async_copy(c_hbm.at[u_idx.at[c]], ubufs[bb], sems[bb])
        dp = pltpu.async_copy(c_hbm.at[p_idx.at[c]], pbufs[bb], sems[bb])
        return du, dp

    lanes = lax.iota(jnp.int32, L)

    def compute(c):
        bb = c % 2
        ub, pb = ubufs[bb], pbufs[bb]

        def group(g, carry):
            rows = lanes + g * L
            acc = jnp.zeros((L,), jnp.float32)
            for d in range(D):
                ucol = jnp.full((L,), d, jnp.int32)
                pcol = jnp.full((L,), D + d, jnp.int32)
                ug = plsc.load_gather(ub, [rows, ucol])
                pg = plsc.load_gather(pb, [rows, pcol])
                acc = acc + ug * pg
            res = 1.0 / (1.0 + jnp.exp(-acc))
            out_v[pl.ds(c * CH + g * L, L)] = res
            return carry

        lax.fori_loop(0, CH // L, group, 0)

    pending = [fire(0), fire(1)]
    for c in range(NCHUNK):
        du, dp = pending[c]
        du.wait()
        dp.wait()
        compute(c)
        if c + 2 < NCHUNK:
            pending.append(fire(c + 2))

    pltpu.sync_copy(out_v, out_hbm.at[pl.ds(base, BPW)])


def kernel(u, p, user_table, prod_table):
    combined = jnp.concatenate([user_table, prod_table], axis=1)
    return _two_tower(u, p, combined)
